# Initial kernel scaffold; baseline (speedup 1.0000x reference)
import jax
import jax.numpy as jnp
from jax.experimental import pallas as pl


def kernel(segment_ids, weight):
    raise NotImplementedError("write your pallas kernel here")


# speedup vs baseline: 3.5876x; 3.5876x over previous
import functools

import jax
import jax.numpy as jnp
from jax import lax
from jax.experimental import pallas as pl
from jax.experimental.pallas import tpu as pltpu
from jax.experimental.pallas import tpu_sc as plsc

NC, NS = 2, 16
NW = NC * NS
IB = 128
GPC = 4
CHUNK = IB * GPC


@functools.partial(jax.jit, static_argnums=(2, 3, 4))
def _gather(idx2d, table, nblk, v, d):
    blk_per_w = nblk // NW
    steps = blk_per_w // GPC
    mesh = plsc.VectorSubcoreMesh(
        core_axis_name="c", subcore_axis_name="s", num_cores=NC, num_subcores=NS
    )

    @functools.partial(
        pl.kernel,
        out_type=jax.ShapeDtypeStruct((nblk * IB, d), jnp.float32),
        mesh=mesh,
        scratch_types=[
            pltpu.VMEM((GPC, IB), jnp.int32),
            pltpu.VMEM((CHUNK, d), jnp.float32),
            pltpu.SemaphoreType.DMA,
        ],
        compiler_params=pltpu.CompilerParams(use_tc_tiling_on_sc=False),
    )
    def k(idx_hbm, table_hbm, out_hbm, idx_v, rows_v, sem):
        wid = lax.axis_index("s") * NC + lax.axis_index("c")
        base_blk = wid * blk_per_w

        def step(g, carry):
            blk = base_blk + g * GPC
            pltpu.sync_copy(idx_hbm.at[pl.ds(blk, GPC)], idx_v)
            copies = []
            for j in range(GPC):
                copies.append(
                    pltpu.async_copy(
                        table_hbm.at[idx_v.at[j]],
                        rows_v.at[pl.ds(j * IB, IB)],
                        sem,
                    )
                )
            for c in copies:
                c.wait()
            pltpu.sync_copy(rows_v, out_hbm.at[pl.ds(blk * IB, CHUNK)])
            return carry

        lax.fori_loop(0, steps, step, 0)

    return k(idx2d, table)


def kernel(segment_ids, weight):
    b, t = segment_ids.shape
    v, d = weight.shape
    n = b * t
    idx2d = segment_ids.reshape(n // IB, IB).astype(jnp.int32)
    out = _gather(idx2d, weight, n // IB, v, d)
    return out.reshape(b, t, d)


# speedup vs baseline: 3.6104x; 1.0064x over previous
import functools

import jax
import jax.numpy as jnp
from jax import lax
from jax.experimental import pallas as pl
from jax.experimental.pallas import tpu as pltpu
from jax.experimental.pallas import tpu_sc as plsc

NC, NS = 2, 16
NW = NC * NS
IB = 128
GPC = 5
CHUNK = IB * GPC


@functools.partial(jax.jit, static_argnums=(2, 3, 4))
def _gather(idx2d, table, nblk, v, d):
    blk_per_w = nblk // NW
    steps = blk_per_w // GPC
    assert steps % 2 == 0 and steps >= 4
    mesh = plsc.VectorSubcoreMesh(
        core_axis_name="c", subcore_axis_name="s", num_cores=NC, num_subcores=NS
    )

    @functools.partial(
        pl.kernel,
        out_type=jax.ShapeDtypeStruct((nblk * IB, d), jnp.float32),
        mesh=mesh,
        scratch_types=[
            pltpu.VMEM((GPC, IB), jnp.int32),
            pltpu.VMEM((GPC, IB), jnp.int32),
            pltpu.VMEM((CHUNK, d), jnp.float32),
            pltpu.VMEM((CHUNK, d), jnp.float32),
            pltpu.SemaphoreType.DMA,
            pltpu.SemaphoreType.DMA,
            pltpu.SemaphoreType.DMA,
            pltpu.SemaphoreType.DMA,
            pltpu.SemaphoreType.DMA,
            pltpu.SemaphoreType.DMA,
        ],
        compiler_params=pltpu.CompilerParams(use_tc_tiling_on_sc=False),
    )
    def k(idx_hbm, table_hbm, out_hbm,
          idx0, idx1, rows0, rows1, si0, si1, sg0, sg1, so0, so1):
        wid = lax.axis_index("s") * NC + lax.axis_index("c")
        base_blk = wid * blk_per_w
        bufs = ((idx0, rows0, si0, sg0, so0), (idx1, rows1, si1, sg1, so1))

        def issue_idx(g, b):
            idx_v, _, si, _, _ = bufs[b]
            pltpu.async_copy(idx_hbm.at[pl.ds(base_blk + g * GPC, GPC)], idx_v, si)

        def run_chunk(g, b, wait_out, next_idx):
            idx_v, rows_v, si, sg, so = bufs[b]
            pltpu.make_async_copy(idx_hbm.at[pl.ds(0, GPC)], idx_v, si).wait()
            if wait_out:
                pltpu.make_async_copy(
                    rows_v, out_hbm.at[pl.ds(0, CHUNK)], so).wait()
            copies = [
                pltpu.async_copy(
                    table_hbm.at[idx_v.at[j]], rows_v.at[pl.ds(j * IB, IB)], sg)
                for j in range(GPC)
            ]
            for c in copies:
                c.wait()
            if next_idx:
                issue_idx(g + 2, b)
            pltpu.async_copy(
                rows_v, out_hbm.at[pl.ds((base_blk + g * GPC) * IB, CHUNK)], so)

        issue_idx(0, 0)
        issue_idx(1, 1)
        run_chunk(0, 0, wait_out=False, next_idx=True)
        run_chunk(1, 1, wait_out=False, next_idx=True)

        def outer(o, carry):
            g = o * 2
            run_chunk(g, 0, wait_out=True, next_idx=True)
            run_chunk(g + 1, 1, wait_out=True, next_idx=True)
            return carry

        lax.fori_loop(1, steps // 2 - 1, outer, 0)

        run_chunk(steps - 2, 0, wait_out=True, next_idx=False)
        run_chunk(steps - 1, 1, wait_out=True, next_idx=False)
        for b in (0, 1):
            _, rows_v, _, _, so = bufs[b]
            pltpu.make_async_copy(rows_v, out_hbm.at[pl.ds(0, CHUNK)], so).wait()

    return k(idx2d, table)


def kernel(segment_ids, weight):
    b, t = segment_ids.shape
    v, d = weight.shape
    n = b * t
    idx2d = segment_ids.reshape(n // IB, IB).astype(jnp.int32)
    out = _gather(idx2d, weight, n // IB, v, d)
    return out.reshape(b, t, d)


# speedup vs baseline: 5.0018x; 1.3854x over previous
import functools

import jax
import jax.numpy as jnp
from jax import lax
from jax.experimental import pallas as pl
from jax.experimental.pallas import tpu as pltpu
from jax.experimental.pallas import tpu_sc as plsc

NC, NS = 2, 16
NW = NC * NS
IB = 128
GPC = 5
CHUNK = IB * GPC


@functools.partial(jax.jit, static_argnums=(2, 3, 4))
def _gather(idx2d, table, nblk, v, d):
    blk_per_w = nblk // NW
    steps = blk_per_w // GPC
    assert steps % 2 == 0 and steps >= 4
    mesh = plsc.VectorSubcoreMesh(
        core_axis_name="c", subcore_axis_name="s", num_cores=NC, num_subcores=NS
    )

    @functools.partial(
        pl.kernel,
        out_type=jax.ShapeDtypeStruct((nblk * IB, d), jnp.float32),
        mesh=mesh,
        scratch_types=[
            pltpu.VMEM_SHARED((v, d), jnp.float32),
            pltpu.VMEM((GPC, IB), jnp.int32),
            pltpu.VMEM((GPC, IB), jnp.int32),
            pltpu.VMEM((CHUNK, d), jnp.float32),
            pltpu.VMEM((CHUNK, d), jnp.float32),
            pltpu.SemaphoreType.DMA,
            pltpu.SemaphoreType.DMA,
            pltpu.SemaphoreType.DMA,
            pltpu.SemaphoreType.DMA,
            pltpu.SemaphoreType.DMA,
            pltpu.SemaphoreType.DMA,
        ],
        compiler_params=pltpu.CompilerParams(use_tc_tiling_on_sc=False),
    )
    def k(idx_hbm, table_hbm, out_hbm,
          table_sh, idx0, idx1, rows0, rows1, si0, si1, sg0, sg1, so0, so1):
        sid = lax.axis_index("s")
        wid = sid * NC + lax.axis_index("c")
        base_blk = wid * blk_per_w
        bufs = ((idx0, rows0, si0, sg0, so0), (idx1, rows1, si1, sg1, so1))

        @pl.when(sid == 0)
        def _():
            pltpu.sync_copy(table_hbm, table_sh)

        plsc.subcore_barrier()

        def issue_idx(g, b):
            idx_v, _, si, _, _ = bufs[b]
            pltpu.async_copy(idx_hbm.at[pl.ds(base_blk + g * GPC, GPC)], idx_v, si)

        def run_chunk(g, b, wait_out, next_idx):
            idx_v, rows_v, si, sg, so = bufs[b]
            pltpu.make_async_copy(idx_hbm.at[pl.ds(0, GPC)], idx_v, si).wait()
            if wait_out:
                pltpu.make_async_copy(
                    rows_v, out_hbm.at[pl.ds(0, CHUNK)], so).wait()
            copies = [
                pltpu.async_copy(
                    table_sh.at[idx_v.at[j]], rows_v.at[pl.ds(j * IB, IB)], sg)
                for j in range(GPC)
            ]
            for c in copies:
                c.wait()
            if next_idx:
                issue_idx(g + 2, b)
            pltpu.async_copy(
                rows_v, out_hbm.at[pl.ds((base_blk + g * GPC) * IB, CHUNK)], so)

        issue_idx(0, 0)
        issue_idx(1, 1)
        run_chunk(0, 0, wait_out=False, next_idx=True)
        run_chunk(1, 1, wait_out=False, next_idx=True)

        def outer(o, carry):
            g = o * 2
            run_chunk(g, 0, wait_out=True, next_idx=True)
            run_chunk(g + 1, 1, wait_out=True, next_idx=True)
            return carry

        lax.fori_loop(1, steps // 2 - 1, outer, 0)

        run_chunk(steps - 2, 0, wait_out=True, next_idx=False)
        run_chunk(steps - 1, 1, wait_out=True, next_idx=False)
        for b in (0, 1):
            _, rows_v, _, _, so = bufs[b]
            pltpu.make_async_copy(rows_v, out_hbm.at[pl.ds(0, CHUNK)], so).wait()

    return k(idx2d, table)


def kernel(segment_ids, weight):
    b, t = segment_ids.shape
    v, d = weight.shape
    n = b * t
    idx2d = segment_ids.reshape(n // IB, IB).astype(jnp.int32)
    out = _gather(idx2d, weight, n // IB, v, d)
    return out.reshape(b, t, d)
